# baseline (speedup 1.0000x reference)
import functools

import jax
import jax.numpy as jnp
from jax import lax
from jax.experimental import pallas as pl
from jax.experimental.pallas import tpu as pltpu
from jax.experimental.pallas import tpu_sc as plsc

NUM_CORES = 2
NUM_SUBCORES = 16
LANES = 16
NW = NUM_CORES * NUM_SUBCORES

EMBED_DIM = 128
WORDS = EMBED_DIM // 2
CHUNK = 64
NBUF = 4


def _sc_body(n_chunks, n_markers, bins_hbm, markers_hbm, bin_tab, mark_pk,
             out_hbm, bidx, midx, bbuf, mtab, sem_ib, sem_im, sem_gb, sem_st):
    wid = lax.axis_index("s") * NUM_CORES + lax.axis_index("c")
    per_w = n_chunks * CHUNK
    w_base = wid * per_w

    pltpu.sync_copy(mark_pk, mtab)

    def issue_idx(g, b):
        base = w_base + g * CHUNK
        pltpu.async_copy(bins_hbm.at[pl.ds(base, CHUNK)], bidx[b], sem_ib[b])
        pltpu.async_copy(markers_hbm.at[pl.ds(base, CHUNK)], midx[b], sem_im[b])

    def wait_idx(g, b):
        base = w_base + g * CHUNK
        pltpu.make_async_copy(bins_hbm.at[pl.ds(base, CHUNK)], bidx[b],
                              sem_ib[b]).wait()
        pltpu.make_async_copy(markers_hbm.at[pl.ds(base, CHUNK)], midx[b],
                              sem_im[b]).wait()

    def issue_gather(b):
        pltpu.async_copy(bin_tab.at[bidx[b]], bbuf[b], sem_gb[b])

    def wait_gather(b):
        pltpu.make_async_copy(bin_tab.at[bidx[b]], bbuf[b], sem_gb[b]).wait()

    def wait_store(g, b):
        base = w_base + g * CHUNK
        pltpu.make_async_copy(bbuf[b], out_hbm.at[pl.ds(base, CHUNK)],
                              sem_st[b]).wait()

    def add_markers(b):
        for gi in range(CHUNK // LANES):

            def col_body(c, _, gi=gi):
                rows = lax.iota(jnp.int32, LANES) + (gi * LANES)
                mrows = midx[b][pl.ds(gi * LANES, LANES)]
                w = plsc.load_gather(mtab, [mrows * WORDS + c])
                lo, hi = plsc.unpack(plsc.bitcast(w, jnp.bfloat16),
                                     format=plsc.PackFormat.INTERLEAVED)
                plsc.addupdate_scatter(bbuf[b],
                                       [rows, jnp.full((LANES,), 2 * c,
                                                       jnp.int32)], lo)
                plsc.addupdate_scatter(bbuf[b],
                                       [rows, jnp.full((LANES,), 2 * c + 1,
                                                       jnp.int32)], hi)
                return 0

            lax.fori_loop(0, WORDS, col_body, 0, unroll=4)

    for b in range(NBUF):
        issue_idx(b, b)
    wait_idx(0, 0)
    issue_gather(0)

    def outer(o, _):
        for b in range(NBUF):
            t = o * NBUF + b
            wait_gather(b)
            add_markers(b)

            st_base = w_base + t * CHUNK
            pltpu.async_copy(bbuf[b], out_hbm.at[pl.ds(st_base, CHUNK)],
                             sem_st[b])

            @pl.when(t + NBUF < n_chunks)
            def _prefetch_idx():
                issue_idx(t + NBUF, b)

            b1 = (b + 1) % NBUF
            nt = t + 1

            @pl.when(nt < n_chunks)
            def _next_gather():
                @pl.when(nt >= NBUF)
                def _drain():
                    wait_store(nt - NBUF, b1)

                wait_idx(nt, b1)
                issue_gather(b1)
        return 0

    lax.fori_loop(0, n_chunks // NBUF, outer, 0, unroll=False)

    for b in range(NBUF):
        wait_store(n_chunks - NBUF + b, (n_chunks - NBUF + b) % NBUF)


@jax.jit
def kernel(bins, markers, bin_table, marker_table):
    b, f = bins.shape
    total = b * f
    n_chunks = total // (NW * CHUNK)
    bins_flat = bins.T.reshape(total).astype(jnp.int32)
    markers_flat = markers.T.reshape(total).astype(jnp.int32)
    n_markers = marker_table.shape[0]
    mark_pk = lax.bitcast_convert_type(
        marker_table.astype(jnp.bfloat16).reshape(n_markers, WORDS, 2),
        jnp.int32).reshape(n_markers * WORDS)

    mesh = plsc.VectorSubcoreMesh(core_axis_name="c", subcore_axis_name="s")
    run = pl.kernel(
        functools.partial(_sc_body, n_chunks, n_markers),
        out_type=jax.ShapeDtypeStruct((total, EMBED_DIM), jnp.float32),
        mesh=mesh,
        compiler_params=pltpu.CompilerParams(needs_layout_passes=False),
        scratch_types=[
            [pltpu.VMEM((CHUNK,), jnp.int32) for _ in range(NBUF)],
            [pltpu.VMEM((CHUNK,), jnp.int32) for _ in range(NBUF)],
            [pltpu.VMEM((CHUNK, EMBED_DIM), jnp.float32) for _ in range(NBUF)],
            pltpu.VMEM((n_markers * WORDS,), jnp.int32),
            [pltpu.SemaphoreType.DMA for _ in range(NBUF)],
            [pltpu.SemaphoreType.DMA for _ in range(NBUF)],
            [pltpu.SemaphoreType.DMA for _ in range(NBUF)],
            [pltpu.SemaphoreType.DMA for _ in range(NBUF)],
        ],
    )
    out = run(bins_flat, markers_flat, bin_table, mark_pk)
    return out.reshape(f, b, bin_table.shape[1]).transpose(1, 0, 2)


# speedup vs baseline: 5.5305x; 5.5305x over previous
import functools

import jax
import jax.numpy as jnp
from jax import lax
from jax.experimental import pallas as pl
from jax.experimental.pallas import tpu as pltpu
from jax.experimental.pallas import tpu_sc as plsc

NUM_CORES = 2
NUM_SUBCORES = 16
LANES = 16
NW = NUM_CORES * NUM_SUBCORES

EMBED_DIM = 128
CHUNK = 64
NBUF = 5


def _sc_body(n_chunks, n_markers, bins_hbm, markers_hbm, bin_tab, mark_tab,
             out_hbm, bidx, midx, bbuf, mbuf, shared_mt, sem_ib, sem_im,
             sem_gb, sem_gm, sem_st):
    sid = lax.axis_index("s")
    wid = sid * NUM_CORES + lax.axis_index("c")
    per_w = n_chunks * CHUNK
    w_base = wid * per_w

    rows_even = (-(-n_markers // NUM_SUBCORES) + 7) // 8 * 8
    n_full, rows_last = divmod(n_markers, rows_even)

    @pl.when(sid < n_full)
    def _stage_full():
        s = pl.ds(sid * rows_even, rows_even)
        pltpu.sync_copy(mark_tab.at[s], shared_mt.at[s])

    if rows_last:
        @pl.when(sid == n_full)
        def _stage_tail():
            s = pl.ds(n_full * rows_even, rows_last)
            pltpu.sync_copy(mark_tab.at[s], shared_mt.at[s])

    plsc.subcore_barrier()

    def issue_idx(g, b):
        base = w_base + g * CHUNK
        pltpu.async_copy(bins_hbm.at[pl.ds(base, CHUNK)], bidx[b], sem_ib[b])
        pltpu.async_copy(markers_hbm.at[pl.ds(base, CHUNK)], midx[b], sem_im[b])

    def wait_idx(g, b):
        base = w_base + g * CHUNK
        pltpu.make_async_copy(bins_hbm.at[pl.ds(base, CHUNK)], bidx[b],
                              sem_ib[b]).wait()
        pltpu.make_async_copy(markers_hbm.at[pl.ds(base, CHUNK)], midx[b],
                              sem_im[b]).wait()

    def issue_gathers(b):
        pltpu.async_copy(bin_tab.at[bidx[b]], bbuf[b], sem_gb[b])
        pltpu.async_copy(shared_mt.at[midx[b]], mbuf[b], sem_gm[b])

    def wait_gathers(b):
        pltpu.make_async_copy(bin_tab.at[bidx[b]], bbuf[b], sem_gb[b]).wait()
        pltpu.make_async_copy(shared_mt.at[midx[b]], mbuf[b], sem_gm[b]).wait()

    def wait_store(g, b):
        base = w_base + g * CHUNK
        pltpu.make_async_copy(bbuf[b], out_hbm.at[pl.ds(base, CHUNK)],
                              sem_st[b]).wait()

    for b in range(NBUF):
        issue_idx(b, b)
    wait_idx(0, 0)
    issue_gathers(0)

    def outer(o, _):
        for b in range(NBUF):
            t = o * NBUF + b
            wait_gathers(b)

            def row_body(r, _):
                for j in range(EMBED_DIM // LANES):
                    s = pl.ds(j * LANES, LANES)
                    bbuf[b][r, s] = bbuf[b][r, s] + mbuf[b][r, s]
                return 0

            lax.fori_loop(0, CHUNK, row_body, 0, unroll=False)

            st_base = w_base + t * CHUNK
            pltpu.async_copy(bbuf[b], out_hbm.at[pl.ds(st_base, CHUNK)],
                             sem_st[b])

            @pl.when(t + NBUF < n_chunks)
            def _prefetch_idx():
                issue_idx(t + NBUF, b)

            b1 = (b + 1) % NBUF
            nt = t + 1

            @pl.when(nt < n_chunks)
            def _next_gather():
                @pl.when(nt >= NBUF)
                def _drain():
                    wait_store(nt - NBUF, b1)

                wait_idx(nt, b1)
                issue_gathers(b1)
        return 0

    lax.fori_loop(0, n_chunks // NBUF, outer, 0, unroll=False)

    for t in range(n_chunks - NBUF, n_chunks):
        wait_store(t, t % NBUF)


@jax.jit
def kernel(bins, markers, bin_table, marker_table):
    b, f = bins.shape
    total = b * f
    n_chunks = total // (NW * CHUNK)
    bins_flat = bins.T.reshape(total).astype(jnp.int32)
    markers_flat = markers.T.reshape(total).astype(jnp.int32)

    mesh = plsc.VectorSubcoreMesh(core_axis_name="c", subcore_axis_name="s")
    run = pl.kernel(
        functools.partial(_sc_body, n_chunks, marker_table.shape[0]),
        out_type=jax.ShapeDtypeStruct((total, EMBED_DIM), jnp.float32),
        mesh=mesh,
        scratch_types=[
            [pltpu.VMEM((CHUNK,), jnp.int32) for _ in range(NBUF)],
            [pltpu.VMEM((CHUNK,), jnp.int32) for _ in range(NBUF)],
            [pltpu.VMEM((CHUNK, EMBED_DIM), jnp.float32) for _ in range(NBUF)],
            [pltpu.VMEM((CHUNK, EMBED_DIM), jnp.float32) for _ in range(NBUF)],
            pltpu.VMEM_SHARED((marker_table.shape[0], EMBED_DIM), jnp.float32),
            [pltpu.SemaphoreType.DMA for _ in range(NBUF)],
            [pltpu.SemaphoreType.DMA for _ in range(NBUF)],
            [pltpu.SemaphoreType.DMA for _ in range(NBUF)],
            [pltpu.SemaphoreType.DMA for _ in range(NBUF)],
            [pltpu.SemaphoreType.DMA for _ in range(NBUF)],
        ],
    )
    out = run(bins_flat, markers_flat, bin_table, marker_table)
    return out.reshape(f, b, bin_table.shape[1]).transpose(1, 0, 2)


# speedup vs baseline: 11.7102x; 2.1174x over previous
import functools

import jax
import jax.numpy as jnp
from jax import lax
from jax.experimental import pallas as pl
from jax.experimental.pallas import tpu as pltpu
from jax.experimental.pallas import tpu_sc as plsc

NUM_CORES = 2
NUM_SUBCORES = 16
LANES = 16
NW = NUM_CORES * NUM_SUBCORES

EMBED_DIM = 128
CHUNK = 64
NBUF = 4


def _sc_body(n_chunks, n_markers, bins_hbm, markers_hbm, bin_tab, mark_tab,
             out_hbm, bidx, midx, bbuf, mbuf, obuf, shared_mt, sem_ib, sem_im,
             sem_gb, sem_gm, sem_st):
    sid = lax.axis_index("s")
    wid = sid * NUM_CORES + lax.axis_index("c")
    per_w = n_chunks * CHUNK
    w_base = wid * per_w

    rows_even = (-(-n_markers // NUM_SUBCORES) + 7) // 8 * 8
    n_full, rows_last = divmod(n_markers, rows_even)

    @pl.when(sid < n_full)
    def _stage_full():
        s = pl.ds(sid * rows_even, rows_even)
        pltpu.sync_copy(mark_tab.at[s], shared_mt.at[s])

    if rows_last:
        @pl.when(sid == n_full)
        def _stage_tail():
            s = pl.ds(n_full * rows_even, rows_last)
            pltpu.sync_copy(mark_tab.at[s], shared_mt.at[s])

    plsc.subcore_barrier()

    def issue_idx(g, b):
        base = w_base + g * CHUNK
        pltpu.async_copy(bins_hbm.at[pl.ds(base, CHUNK)], bidx[b], sem_ib[b])
        pltpu.async_copy(markers_hbm.at[pl.ds(base, CHUNK)], midx[b], sem_im[b])

    def wait_idx(g, b):
        base = w_base + g * CHUNK
        pltpu.make_async_copy(bins_hbm.at[pl.ds(base, CHUNK)], bidx[b],
                              sem_ib[b]).wait()
        pltpu.make_async_copy(markers_hbm.at[pl.ds(base, CHUNK)], midx[b],
                              sem_im[b]).wait()

    def issue_gathers(b):
        pltpu.async_copy(bin_tab.at[bidx[b]], bbuf[b], sem_gb[b])
        pltpu.async_copy(shared_mt.at[midx[b]], mbuf[b], sem_gm[b])

    def wait_gathers(b):
        pltpu.make_async_copy(bin_tab.at[bidx[b]], bbuf[b], sem_gb[b]).wait()
        pltpu.make_async_copy(shared_mt.at[midx[b]], mbuf[b], sem_gm[b]).wait()

    def wait_store(g, b):
        base = w_base + g * CHUNK
        pltpu.make_async_copy(obuf[b], out_hbm.at[pl.ds(base, CHUNK)],
                              sem_st[b]).wait()

    for b in range(NBUF):
        issue_idx(b, b)
        wait_idx(b, b)
        issue_gathers(b)

    def outer(o, _):
        for b in range(NBUF):
            g = o * NBUF + b
            ng = g + NBUF
            wait_gathers(b)

            @pl.when(ng < n_chunks)
            def _prefetch_idx():
                issue_idx(ng, b)

            @pl.when(g >= NBUF)
            def _drain_old_store():
                wait_store(g - NBUF, b)

            def row_body(r, _):
                for j in range(EMBED_DIM // LANES):
                    s = pl.ds(j * LANES, LANES)
                    obuf[b][r, s] = bbuf[b][r, s] + mbuf[b][r, s]
                return 0

            lax.fori_loop(0, CHUNK, row_body, 0, unroll=False)

            st_base = w_base + g * CHUNK
            pltpu.async_copy(obuf[b], out_hbm.at[pl.ds(st_base, CHUNK)],
                             sem_st[b])

            @pl.when(ng < n_chunks)
            def _prefetch_gather():
                wait_idx(ng, b)
                issue_gathers(b)
        return 0

    lax.fori_loop(0, n_chunks // NBUF, outer, 0, unroll=False)

    for b in range(NBUF):
        wait_store(n_chunks - NBUF + b, b)


@jax.jit
def kernel(bins, markers, bin_table, marker_table):
    b, f = bins.shape
    total = b * f
    n_chunks = total // (NW * CHUNK)
    bins_flat = bins.T.reshape(total).astype(jnp.int32)
    markers_flat = markers.T.reshape(total).astype(jnp.int32)

    mesh = plsc.VectorSubcoreMesh(core_axis_name="c", subcore_axis_name="s")
    run = pl.kernel(
        functools.partial(_sc_body, n_chunks, marker_table.shape[0]),
        out_type=jax.ShapeDtypeStruct((total, EMBED_DIM), jnp.float32),
        mesh=mesh,
        scratch_types=[
            [pltpu.VMEM((CHUNK,), jnp.int32) for _ in range(NBUF)],
            [pltpu.VMEM((CHUNK,), jnp.int32) for _ in range(NBUF)],
            [pltpu.VMEM((CHUNK, EMBED_DIM), jnp.float32) for _ in range(NBUF)],
            [pltpu.VMEM((CHUNK, EMBED_DIM), jnp.float32) for _ in range(NBUF)],
            [pltpu.VMEM((CHUNK, EMBED_DIM), jnp.float32) for _ in range(NBUF)],
            pltpu.VMEM_SHARED((marker_table.shape[0], EMBED_DIM), jnp.float32),
            [pltpu.SemaphoreType.DMA for _ in range(NBUF)],
            [pltpu.SemaphoreType.DMA for _ in range(NBUF)],
            [pltpu.SemaphoreType.DMA for _ in range(NBUF)],
            [pltpu.SemaphoreType.DMA for _ in range(NBUF)],
            [pltpu.SemaphoreType.DMA for _ in range(NBUF)],
        ],
    )
    out = run(bins_flat, markers_flat, bin_table, marker_table)
    return out.reshape(f, b, bin_table.shape[1]).transpose(1, 0, 2)
